# Initial kernel scaffold; baseline (speedup 1.0000x reference)
import jax
import jax.numpy as jnp
from jax.experimental import pallas as pl


def kernel(y_true, y_pred):
    raise NotImplementedError("write your pallas kernel here")


# speedup vs baseline: 17.8004x; 17.8004x over previous
import functools

import jax
import jax.numpy as jnp
from jax import lax
from jax.experimental import pallas as pl
from jax.experimental.pallas import tpu as pltpu
from jax.experimental.pallas import tpu_sc as plsc

_BATCH = 16384
_NC = 2
_NS = 16
_L = 16
_NW = _NC * _NS
_PER_TILE = _BATCH // _NW


def _recall_body(yt_hbm, yp_hbm, out_hbm, yt_v, yp_v, acc_v):
    c = lax.axis_index("c")
    s = lax.axis_index("s")
    wid = s * _NC + c
    base = wid * _PER_TILE

    pltpu.sync_copy(yt_hbm.at[pl.ds(base, _PER_TILE)], yt_v)
    pltpu.sync_copy(yp_hbm.at[pl.ds(base, _PER_TILE)], yp_v)

    def body(i, acc):
        a = yt_v[pl.ds(i * _L, _L)]
        b = yp_v[pl.ds(i * _L, _L)]
        return acc + jnp.where(a == b, jnp.int32(1), jnp.int32(0))

    acc = lax.fori_loop(0, _PER_TILE // _L, body, jnp.zeros((_L,), jnp.int32))

    tot = acc[0]
    for t in range(1, _L):
        tot = tot + acc[t]
    acc_v[...] = jnp.full((_L,), tot, jnp.int32).astype(jnp.float32)
    pltpu.sync_copy(acc_v, out_hbm.at[wid])


@jax.jit
def _recall_call(y_true, y_pred):
    mesh = plsc.VectorSubcoreMesh(core_axis_name="c", subcore_axis_name="s")
    counts = pl.kernel(
        _recall_body,
        out_type=jax.ShapeDtypeStruct((_NW, _L), jnp.float32),
        mesh=mesh,
        scratch_types=[
            pltpu.VMEM((_PER_TILE,), jnp.int32),
            pltpu.VMEM((_PER_TILE,), jnp.int32),
            pltpu.VMEM((_L,), jnp.float32),
        ],
    )(y_true, y_pred)
    return jnp.sum(counts[:, 0]) * (1.0 / _BATCH)


def kernel(y_true, y_pred):
    return _recall_call(y_true.astype(jnp.int32), y_pred.astype(jnp.int32))


# speedup vs baseline: 18.5452x; 1.0418x over previous
import functools

import jax
import jax.numpy as jnp
from jax import lax
from jax.experimental import pallas as pl
from jax.experimental.pallas import tpu as pltpu
from jax.experimental.pallas import tpu_sc as plsc

_BATCH = 16384
_NS = 16
_L = 16
_PER_TILE = _BATCH // _NS


def _recall_body(yt_hbm, yp_hbm, out_hbm, yt_v, yp_v, acc_v, gather_v, sem1, sem2):
    s = lax.axis_index("s")
    base = s * _PER_TILE

    cp1 = pltpu.async_copy(yt_hbm.at[pl.ds(base, _PER_TILE)], yt_v, sem1)
    cp2 = pltpu.async_copy(yp_hbm.at[pl.ds(base, _PER_TILE)], yp_v, sem2)
    cp1.wait()
    cp2.wait()

    def body(i, acc):
        a = yt_v[pl.ds(i * _L, _L)]
        b = yp_v[pl.ds(i * _L, _L)]
        return acc + jnp.where(a == b, jnp.int32(1), jnp.int32(0))

    acc = lax.fori_loop(0, _PER_TILE // _L, body, jnp.zeros((_L,), jnp.int32))
    acc_v[...] = acc.astype(jnp.float32)
    pltpu.sync_copy(acc_v, out_hbm.at[s])
    plsc.subcore_barrier()

    @pl.when(s == 0)
    def _():
        pltpu.sync_copy(out_hbm, gather_v)
        tot = gather_v[0]
        for t in range(1, _NS):
            tot = tot + gather_v[t]
        sc = tot[0]
        for t in range(1, _L):
            sc = sc + tot[t]
        acc_v[...] = jnp.full((_L,), sc, jnp.float32)
        pltpu.sync_copy(acc_v, out_hbm.at[0])


@jax.jit
def _recall_call(y_true, y_pred):
    mesh = plsc.VectorSubcoreMesh(
        core_axis_name="c", subcore_axis_name="s", num_cores=1
    )
    counts = pl.kernel(
        _recall_body,
        out_type=jax.ShapeDtypeStruct((_NS, _L), jnp.float32),
        mesh=mesh,
        scratch_types=[
            pltpu.VMEM((_PER_TILE,), jnp.int32),
            pltpu.VMEM((_PER_TILE,), jnp.int32),
            pltpu.VMEM((_L,), jnp.float32),
            pltpu.VMEM((_NS, _L), jnp.float32),
            pltpu.SemaphoreType.DMA,
            pltpu.SemaphoreType.DMA,
        ],
    )(y_true, y_pred)
    return counts[0, 0] * (1.0 / _BATCH)


def kernel(y_true, y_pred):
    return _recall_call(y_true.astype(jnp.int32), y_pred.astype(jnp.int32))


# speedup vs baseline: 22.1682x; 1.1954x over previous
import functools

import jax
import jax.numpy as jnp
from jax import lax
from jax.experimental import pallas as pl
from jax.experimental.pallas import tpu as pltpu
from jax.experimental.pallas import tpu_sc as plsc

_BATCH = 16384
_NS = 16
_L = 16
_PER_TILE = _BATCH // _NS


def _recall_body(yt_hbm, yp_hbm, out_hbm, yt_v, yp_v, acc_v, cnt_smem, sem1, sem2):
    s = lax.axis_index("s")
    base = s * _PER_TILE

    cp1 = pltpu.async_copy(yt_hbm.at[pl.ds(base, _PER_TILE)], yt_v, sem1)
    cp2 = pltpu.async_copy(yp_hbm.at[pl.ds(base, _PER_TILE)], yp_v, sem2)

    @pl.when(s == 0)
    def _():
        cnt_smem[0] = jnp.int32(0)

    plsc.subcore_barrier()
    cp1.wait()
    cp2.wait()

    def body(i, acc):
        a = yt_v[pl.ds(i * _L, _L)]
        b = yp_v[pl.ds(i * _L, _L)]
        return acc + jnp.where(a == b, jnp.int32(1), jnp.int32(0))

    acc = lax.fori_loop(
        0, _PER_TILE // _L, body, jnp.zeros((_L,), jnp.int32), unroll=8
    )
    sc = acc[0]
    for t in range(1, _L):
        sc = sc + acc[t]
    plsc.fetch_and_add(cnt_smem.at[0], sc, subcore_id=0)
    plsc.subcore_barrier()

    @pl.when(s == 0)
    def _():
        total = cnt_smem[0]
        acc_v[...] = (
            jnp.full((_L,), total, jnp.int32).astype(jnp.float32) * (1.0 / _BATCH)
        )
        pltpu.sync_copy(acc_v, out_hbm)


@jax.jit
def _recall_call(y_true, y_pred):
    mesh = plsc.VectorSubcoreMesh(
        core_axis_name="c", subcore_axis_name="s", num_cores=1
    )
    out = pl.kernel(
        _recall_body,
        out_type=jax.ShapeDtypeStruct((_L,), jnp.float32),
        mesh=mesh,
        scratch_types=[
            pltpu.VMEM((_PER_TILE,), jnp.int32),
            pltpu.VMEM((_PER_TILE,), jnp.int32),
            pltpu.VMEM((_L,), jnp.float32),
            pltpu.SMEM((1,), jnp.int32),
            pltpu.SemaphoreType.DMA,
            pltpu.SemaphoreType.DMA,
        ],
    )(y_true, y_pred)
    return out[0]


def kernel(y_true, y_pred):
    return _recall_call(y_true.astype(jnp.int32), y_pred.astype(jnp.int32))
